# baseline (speedup 1.0000x reference)
import functools

import jax
import jax.numpy as jnp
from jax import lax
from jax.experimental import pallas as pl
from jax.experimental.pallas import tpu as pltpu
from jax.experimental.pallas import tpu_sc as plsc

_NUM_CORES = 2
_NUM_SUBCORES = 16
_NW = _NUM_CORES * _NUM_SUBCORES
_CHUNK = 128


@functools.partial(jax.jit, static_argnums=(2,))
def _sc_gather(idx2d, table, n_rows_per_w):
    n_total = idx2d.shape[0] * _CHUNK
    d = table.shape[1]
    mesh = plsc.VectorSubcoreMesh(core_axis_name="c", subcore_axis_name="s")

    @functools.partial(
        pl.kernel,
        mesh=mesh,
        out_type=jax.ShapeDtypeStruct((n_total, d), jnp.float32),
        scratch_types=[
            pltpu.VMEM((n_rows_per_w, _CHUNK), jnp.int32),
            pltpu.VMEM((_CHUNK, d), jnp.float32),
            pltpu.SemaphoreType.DMA,
        ],
        compiler_params=pltpu.CompilerParams(use_tc_tiling_on_sc=False),
    )
    def k(idx_hbm, table_hbm, out_hbm, idx_v, rows_v, sem):
        wid = lax.axis_index("s") * _NUM_CORES + lax.axis_index("c")
        row0 = wid * n_rows_per_w
        pltpu.sync_copy(idx_hbm.at[pl.ds(row0, n_rows_per_w)], idx_v)

        def body(j, carry):
            pltpu.async_copy(table_hbm.at[idx_v.at[j]], rows_v, sem).wait()
            pltpu.sync_copy(rows_v, out_hbm.at[pl.ds((row0 + j) * _CHUNK, _CHUNK)])
            return carry

        lax.fori_loop(0, n_rows_per_w, body, 0)

    return k(idx2d, table)


def kernel(indices, table):
    b, h = indices.shape
    v, d = table.shape
    n = b * h
    assert n % (_NW * _CHUNK) == 0
    n_rows_per_w = n // (_NW * _CHUNK)
    idx2d = indices.astype(jnp.int32).reshape(n // _CHUNK, _CHUNK)
    out = _sc_gather(idx2d, table, n_rows_per_w)
    return out.reshape(b, h, d)
